# baseline (speedup 1.0000x reference)
import jax
import jax.numpy as jnp
from jax import lax
from jax.experimental import pallas as pl
from jax.experimental.pallas import tpu as pltpu


def _argmin_body(x_ref, we_ref, be_ref, cb_ref, idx_ref, webf_ref):
    K = cb_ref.shape[0]

    @pl.when(pl.program_id(0) == 0)
    def _():
        webf_ref[...] = we_ref[...].astype(jnp.bfloat16)

    enc = jnp.dot(x_ref[...].astype(jnp.bfloat16), webf_ref[...],
                  preferred_element_type=jnp.float32)
    enc = enc + be_ref[...]
    a = jnp.sum(enc * enc, axis=1, keepdims=True)
    cb32 = cb_ref[...]
    m = lax.dot_general(enc.astype(jnp.bfloat16), cb32.astype(jnp.bfloat16),
                        (((1,), (1,)), ((), ())),
                        preferred_element_type=jnp.float32)
    c2 = jnp.sum(cb32 * cb32, axis=1)
    d = (a - 2.0 * m) + c2[None, :]
    dmin = jnp.min(d, axis=1, keepdims=True)
    ii = lax.broadcasted_iota(jnp.int32, d.shape, 1)
    idx_ref[...] = jnp.min(jnp.where(d == dmin, ii, K), axis=1).astype(jnp.int32)


def _gather_body(idx_ref, cb_ref, wd_ref, bd_ref, out_ref, dcb_ref):
    K = cb_ref.shape[0]

    @pl.when(pl.program_id(1) == 0)
    def _():
        dec = jnp.dot(cb_ref[...].astype(jnp.bfloat16),
                      wd_ref[...].astype(jnp.bfloat16),
                      preferred_element_type=jnp.float32) + bd_ref[...]
        dcb_ref[...] = dec.astype(jnp.bfloat16)

    idxb = idx_ref[...]
    oh = (idxb[:, None] == lax.broadcasted_iota(jnp.int32, (idxb.shape[0], K), 1))
    out_ref[...] = jnp.dot(oh.astype(jnp.bfloat16), dcb_ref[...],
                           preferred_element_type=jnp.float32)


def kernel(inputs, W_enc, b_enc, codebook, W_dec, b_dec):
    B, D_IN = inputs.shape
    K, D_CODE = codebook.shape

    BM = min(256, B)
    BG = min(128, B)
    BJ = min(8192, D_IN)

    indices = pl.pallas_call(
        _argmin_body,
        grid=(B // BM,),
        in_specs=[
            pl.BlockSpec((BM, D_IN), lambda i: (i, 0)),
            pl.BlockSpec((D_IN, D_CODE), lambda i: (0, 0)),
            pl.BlockSpec((1, D_CODE), lambda i: (0, 0)),
            pl.BlockSpec((K, D_CODE), lambda i: (0, 0)),
        ],
        out_specs=pl.BlockSpec((BM,), lambda i: (i,)),
        out_shape=jax.ShapeDtypeStruct((B,), jnp.int32),
        scratch_shapes=[pltpu.VMEM((D_IN, D_CODE), jnp.bfloat16)],
    )(inputs, W_enc, b_enc.reshape(1, D_CODE), codebook)

    out = pl.pallas_call(
        _gather_body,
        grid=(D_IN // BJ, B // BG),
        in_specs=[
            pl.BlockSpec((BG,), lambda j, b: (b,)),
            pl.BlockSpec((K, D_CODE), lambda j, b: (0, 0)),
            pl.BlockSpec((D_CODE, BJ), lambda j, b: (0, j)),
            pl.BlockSpec((1, BJ), lambda j, b: (0, j)),
        ],
        out_specs=pl.BlockSpec((BG, BJ), lambda j, b: (b, j)),
        out_shape=jax.ShapeDtypeStruct((B, D_IN), jnp.float32),
        scratch_shapes=[pltpu.VMEM((K, BJ), jnp.bfloat16)],
    )(indices, codebook, W_dec, b_dec.reshape(1, D_IN))

    return out


# speedup vs baseline: 1.0467x; 1.0467x over previous
import jax
import jax.numpy as jnp
from jax import lax
from jax.experimental import pallas as pl
from jax.experimental.pallas import tpu as pltpu


def _argmin_body(x_ref, we_ref, be_ref, cb_ref, idx_ref, webf_ref):
    K = cb_ref.shape[0]

    @pl.when(pl.program_id(0) == 0)
    def _():
        webf_ref[...] = we_ref[...].astype(jnp.bfloat16)

    enc = jnp.dot(x_ref[...].astype(jnp.bfloat16), webf_ref[...],
                  preferred_element_type=jnp.float32)
    enc = enc + be_ref[...]
    a = jnp.sum(enc * enc, axis=1, keepdims=True)
    cb32 = cb_ref[...]
    m = lax.dot_general(enc.astype(jnp.bfloat16), cb32.astype(jnp.bfloat16),
                        (((1,), (1,)), ((), ())),
                        preferred_element_type=jnp.float32)
    c2 = jnp.sum(cb32 * cb32, axis=1)
    d = (a - 2.0 * m) + c2[None, :]
    dmin = jnp.min(d, axis=1, keepdims=True)
    ii = lax.broadcasted_iota(jnp.int32, d.shape, 1)
    idx_ref[...] = jnp.min(jnp.where(d == dmin, ii, K), axis=1).astype(jnp.int32)


def _gather_body(idx_ref, cb_ref, wd_ref, bd_ref, out_ref, dcb_ref):
    K = cb_ref.shape[0]

    @pl.when(pl.program_id(1) == 0)
    def _():
        dec = jnp.dot(cb_ref[...].astype(jnp.bfloat16),
                      wd_ref[...].astype(jnp.bfloat16),
                      preferred_element_type=jnp.float32) + bd_ref[...]
        dcb_ref[...] = dec.astype(jnp.bfloat16)

    idxb = idx_ref[...]
    oh = (idxb[:, None] == lax.broadcasted_iota(jnp.int32, (idxb.shape[0], K), 1))
    out_ref[...] = jnp.dot(oh.astype(jnp.bfloat16), dcb_ref[...],
                           preferred_element_type=jnp.float32)


def kernel(inputs, W_enc, b_enc, codebook, W_dec, b_dec):
    B, D_IN = inputs.shape
    K, D_CODE = codebook.shape

    BM = min(256, B)
    BG = min(256, B)
    BJ = min(8192, D_IN)

    indices = pl.pallas_call(
        _argmin_body,
        grid=(B // BM,),
        in_specs=[
            pl.BlockSpec((BM, D_IN), lambda i: (i, 0)),
            pl.BlockSpec((D_IN, D_CODE), lambda i: (0, 0)),
            pl.BlockSpec((1, D_CODE), lambda i: (0, 0)),
            pl.BlockSpec((K, D_CODE), lambda i: (0, 0)),
        ],
        out_specs=pl.BlockSpec((BM,), lambda i: (i,)),
        out_shape=jax.ShapeDtypeStruct((B,), jnp.int32),
        scratch_shapes=[pltpu.VMEM((D_IN, D_CODE), jnp.bfloat16)],
    )(inputs, W_enc, b_enc.reshape(1, D_CODE), codebook)

    out = pl.pallas_call(
        _gather_body,
        grid=(D_IN // BJ, B // BG),
        in_specs=[
            pl.BlockSpec((BG,), lambda j, b: (b,)),
            pl.BlockSpec((K, D_CODE), lambda j, b: (0, 0)),
            pl.BlockSpec((D_CODE, BJ), lambda j, b: (0, j)),
            pl.BlockSpec((1, BJ), lambda j, b: (0, j)),
        ],
        out_specs=pl.BlockSpec((BG, BJ), lambda j, b: (b, j)),
        out_shape=jax.ShapeDtypeStruct((B, D_IN), jnp.float32),
        scratch_shapes=[pltpu.VMEM((K, BJ), jnp.bfloat16)],
    )(indices, codebook, W_dec, b_dec.reshape(1, D_IN))

    return out
